# baseline (speedup 1.0000x reference)
import functools

import jax
import jax.numpy as jnp
from jax import lax
from jax.experimental import pallas as pl
from jax.experimental.pallas import tpu as pltpu
from jax.experimental.pallas import tpu_sc as plsc

VOCAB = 100000
DIM = 100
DPAD = 128
SEQ = 200
BATCH = 4096

NC = 2
NS = 16
NW = NC * NS
ROWS = SEQ * BATCH
RPW = ROWS // NW
CH = 128
NCH = RPW // CH
NSLOT = 4

_mesh = plsc.VectorSubcoreMesh(core_axis_name="c", subcore_axis_name="s")


@functools.partial(
    pl.kernel,
    mesh=_mesh,
    out_type=jax.ShapeDtypeStruct((ROWS, DPAD), jnp.float32),
    scratch_types=[
        pltpu.VMEM((CH,), jnp.int32),
        pltpu.VMEM((CH,), jnp.int32),
        pltpu.VMEM((CH,), jnp.int32),
        pltpu.VMEM((CH,), jnp.int32),
        pltpu.VMEM((NSLOT, CH, DPAD), jnp.float32),
        pltpu.SemaphoreType.DMA,
        pltpu.SemaphoreType.DMA,
        pltpu.SemaphoreType.DMA,
        pltpu.SemaphoreType.DMA,
        pltpu.SemaphoreType.DMA,
        pltpu.SemaphoreType.DMA,
        pltpu.SemaphoreType.DMA,
        pltpu.SemaphoreType.DMA,
    ],
)
def _gather_kernel(ids_hbm, table_hbm, out_hbm,
                   i0, i1, i2, i3, rows_v,
                   g0, g1, g2, g3, s0, s1, s2, s3):
    idx = (i0, i1, i2, i3)
    sem_g = (g0, g1, g2, g3)
    sem_s = (s0, s1, s2, s3)
    w = lax.axis_index("s") * NC + lax.axis_index("c")
    base = w * RPW

    def load_idx(j, p):
        pltpu.sync_copy(ids_hbm.at[pl.ds(base + j * CH, CH)], idx[p])

    def gather(j, p):
        return pltpu.make_async_copy(
            table_hbm.at[idx[p]], rows_v.at[p], sem_g[p]
        )

    def store(j, p):
        return pltpu.make_async_copy(
            rows_v.at[p], out_hbm.at[pl.ds(base + j * CH, CH)], sem_s[p]
        )

    for p in range(2):
        load_idx(p, p)
        gather(p, p).start()

    def body(t, carry):
        for p in range(NSLOT):
            j = NSLOT * t + p
            gather(j, p).wait()
            store(j, p).start()
            q = (p + 2) % NSLOT
            jn = j + 2

            @pl.when(jn < NCH)
            def _prefetch():
                @pl.when(j >= 2)
                def _drain_store():
                    store(j - 2, q).wait()

                load_idx(jn, q)
                gather(jn, q).start()

        return carry

    lax.fori_loop(0, NCH // NSLOT, body, 0)

    store(NCH - 2, (NCH - 2) % NSLOT).wait()
    store(NCH - 1, (NCH - 1) % NSLOT).wait()


def kernel(input_ids, table):
    ids_t = jnp.transpose(input_ids, (1, 0)).reshape(ROWS)
    eye_pad = jnp.eye(DIM, DPAD, dtype=jnp.float32)
    table_pad = jnp.dot(table, eye_pad, precision=jax.lax.Precision.HIGHEST)
    padded = _gather_kernel(ids_t.astype(jnp.int32), table_pad)
    return padded.reshape(BATCH, SEQ, DPAD)[:, :, :DIM]


# speedup vs baseline: 1.0158x; 1.0158x over previous
import functools

import jax
import jax.numpy as jnp
from jax import lax
from jax.experimental import pallas as pl
from jax.experimental.pallas import tpu as pltpu
from jax.experimental.pallas import tpu_sc as plsc

VOCAB = 100000
DIM = 100
DPAD = 128
SEQ = 200
BATCH = 4096

NC = 2
NS = 16
NW = NC * NS
ROWS = SEQ * BATCH
RPW = ROWS // NW
CH = 128
NCH = RPW // CH
NSLOT = 5
PF = 3

_mesh = plsc.VectorSubcoreMesh(core_axis_name="c", subcore_axis_name="s")


@functools.partial(
    pl.kernel,
    mesh=_mesh,
    out_type=jax.ShapeDtypeStruct((ROWS, DPAD), jnp.float32),
    scratch_types=(
        [pltpu.VMEM((CH,), jnp.int32)] * NSLOT
        + [pltpu.VMEM((NSLOT, CH, DPAD), jnp.float32)]
        + [pltpu.SemaphoreType.DMA] * (2 * NSLOT)
    ),
)
def _gather_kernel(ids_hbm, table_hbm, out_hbm, *scratch):
    idx = scratch[:NSLOT]
    rows_v = scratch[NSLOT]
    sem_g = scratch[NSLOT + 1:2 * NSLOT + 1]
    sem_s = scratch[2 * NSLOT + 1:]
    w = lax.axis_index("s") * NC + lax.axis_index("c")
    base = w * RPW

    def load_idx(j, p):
        pltpu.sync_copy(ids_hbm.at[pl.ds(base + j * CH, CH)], idx[p])

    def gather(j, p):
        return pltpu.make_async_copy(
            table_hbm.at[idx[p]], rows_v.at[p], sem_g[p]
        )

    def store(j, p):
        return pltpu.make_async_copy(
            rows_v.at[p], out_hbm.at[pl.ds(base + j * CH, CH)], sem_s[p]
        )

    for p in range(PF):
        load_idx(p, p)
        gather(p, p).start()

    def body(t, carry):
        for p in range(NSLOT):
            j = NSLOT * t + p
            gather(j, p).wait()
            store(j, p).start()
            q = (p + PF) % NSLOT
            jn = j + PF

            @pl.when(jn < NCH)
            def _prefetch():
                @pl.when(j >= NSLOT - PF)
                def _drain_store():
                    store(j - (NSLOT - PF), q).wait()

                load_idx(jn, q)
                gather(jn, q).start()

        return carry

    lax.fori_loop(0, NCH // NSLOT, body, 0)

    for j in range(NCH - (NSLOT - PF), NCH):
        store(j, j % NSLOT).wait()


def kernel(input_ids, table):
    ids_t = jnp.transpose(input_ids, (1, 0)).reshape(ROWS)
    eye_pad = jnp.eye(DIM, DPAD, dtype=jnp.float32)
    table_pad = jnp.dot(table, eye_pad, precision=jax.lax.Precision.HIGHEST)
    padded = _gather_kernel(ids_t.astype(jnp.int32), table_pad)
    return padded.reshape(BATCH, SEQ, DPAD)[:, :, :DIM]
